# Initial kernel scaffold; baseline (speedup 1.0000x reference)
import functools

import jax
import jax.numpy as jnp
from jax import lax
from jax.experimental import pallas as pl
from jax.experimental.pallas import tpu as pltpu, tpu_sc as plsc

NC = 2
NS = 16
L = 16
NW = NC * NS

N = 10000
E = 320000
D = 128

EPT = E // NW
G = EPT // L
NPT = N // NS
DEG_R = 640

_mesh = plsc.VectorSubcoreMesh(core_axis_name="c", subcore_axis_name="s")


@functools.partial(
    pl.kernel,
    out_type=jax.ShapeDtypeStruct((NC, DEG_R, L), jnp.float32),
    mesh=_mesh,
    scratch_types=[
        pltpu.VMEM((EPT,), jnp.int32),
        pltpu.VMEM((DEG_R, L), jnp.float32),
        pltpu.VMEM_SHARED((DEG_R, L), jnp.float32),
    ],
)
def _sc_degree(col_hbm, zeros_hbm, out_hbm, col_v, deg_v, acc):
    cid = lax.axis_index("c")
    sid = lax.axis_index("s")
    tid = sid * NC + cid
    pltpu.sync_copy(col_hbm.at[pl.ds(tid * EPT, EPT)], col_v)
    pltpu.sync_copy(zeros_hbm, deg_v)

    @pl.when(sid == 0)
    def _():
        pltpu.sync_copy(zeros_hbm, acc)

    ones = jnp.ones((L,), jnp.float32)

    def body(g, carry):
        c16 = col_v[pl.ds(g * L, L)]
        plsc.addupdate_scatter(deg_v, [c16 // L, c16 % L], ones)
        return carry

    lax.fori_loop(0, G, body, 0)
    plsc.subcore_barrier()

    iota = lax.iota(jnp.int32, L)

    def add_body(k, carry):
        pltpu.sync_copy(deg_v.at[pl.ds(k * L, L)], acc.at[iota + k * L],
                        add=True)
        return carry

    lax.fori_loop(0, DEG_R // L, add_body, 0)
    plsc.subcore_barrier()

    @pl.when(sid == 0)
    def _():
        pltpu.sync_copy(acc, out_hbm.at[cid])


@functools.partial(
    pl.kernel,
    out_type=jax.ShapeDtypeStruct((E,), jnp.float32),
    mesh=_mesh,
    scratch_types=[
        pltpu.VMEM((N,), jnp.float32),
        pltpu.VMEM((EPT,), jnp.int32),
        pltpu.VMEM((EPT,), jnp.int32),
        pltpu.VMEM((EPT,), jnp.float32),
        pltpu.VMEM((EPT,), jnp.float32),
    ],
)
def _sc_edge_w(row_hbm, col_hbm, ew_hbm, dis_hbm, out_hbm,
               dis_v, row_v, col_v, ew_v, w_v):
    cid = lax.axis_index("c")
    sid = lax.axis_index("s")
    tid = sid * NC + cid
    pltpu.sync_copy(dis_hbm, dis_v)
    pltpu.sync_copy(row_hbm.at[pl.ds(tid * EPT, EPT)], row_v)
    pltpu.sync_copy(col_hbm.at[pl.ds(tid * EPT, EPT)], col_v)
    pltpu.sync_copy(ew_hbm.at[pl.ds(tid * EPT, EPT)], ew_v)

    def body(g, carry):
        sl = pl.ds(g * L, L)
        r16 = row_v[sl]
        c16 = col_v[sl]
        a = plsc.load_gather(dis_v, [r16])
        b = plsc.load_gather(dis_v, [c16])
        w_v[sl] = ew_v[sl] * a * b
        return carry

    lax.fori_loop(0, G, body, 0)
    pltpu.sync_copy(w_v, out_hbm.at[pl.ds(tid * EPT, EPT)])


@functools.partial(
    pl.kernel,
    out_type=jax.ShapeDtypeStruct((NC, N, D), jnp.float32),
    mesh=_mesh,
    scratch_types=[
        pltpu.VMEM((EPT,), jnp.int32),
        pltpu.VMEM((EPT,), jnp.int32),
        pltpu.VMEM((EPT,), jnp.float32),
        pltpu.VMEM((L, D), jnp.float32),
        pltpu.VMEM_SHARED((N, D), jnp.float32),
    ],
)
def _sc_aggregate(h_hbm, row_hbm, col_hbm, w_hbm, zeros_hbm, out_hbm,
                  row_v, col_v, w_v, msg_v, acc):
    cid = lax.axis_index("c")
    sid = lax.axis_index("s")
    tid = sid * NC + cid
    pltpu.sync_copy(row_hbm.at[pl.ds(tid * EPT, EPT)], row_v)
    pltpu.sync_copy(col_hbm.at[pl.ds(tid * EPT, EPT)], col_v)
    pltpu.sync_copy(w_hbm.at[pl.ds(tid * EPT, EPT)], w_v)
    pltpu.sync_copy(zeros_hbm.at[pl.ds(sid * NPT, NPT)],
                    acc.at[pl.ds(sid * NPT, NPT)])
    plsc.subcore_barrier()

    def body(g, carry):
        sl = pl.ds(g * L, L)
        r16 = row_v[sl]
        pltpu.sync_copy(h_hbm.at[r16], msg_v)
        for k in range(L):
            wk = plsc.load_gather(w_v, [jnp.full((L,), 0, jnp.int32) + (g * L + k)])
            for j in range(D // L):
                fsl = pl.ds(j * L, L)
                msg_v[k, fsl] = msg_v[k, fsl] * wk
        c16 = col_v[sl]
        pltpu.sync_copy(msg_v, acc.at[c16], add=True)
        return carry

    lax.fori_loop(0, G, body, 0)
    plsc.subcore_barrier()
    pltpu.sync_copy(acc.at[pl.ds(sid * NPT, NPT)],
                    out_hbm.at[cid, pl.ds(sid * NPT, NPT)])


def _tc_update_body(p0, p1, cinv, w, b, out, *, relu):
    aggr = (p0[...] + p1[...]) * cinv[...]
    y = jnp.dot(aggr, w[...], preferred_element_type=jnp.float32) + b[...]
    if relu:
        y = jnp.maximum(y, 0.0)
    out[...] = y


def _tc_update(p0, p1, cnt_inv, wt, b, relu):
    blk = 2000
    grid = (N // blk,)
    body = functools.partial(_tc_update_body, relu=relu)
    return pl.pallas_call(
        body,
        out_shape=jax.ShapeDtypeStruct((N, D), jnp.float32),
        grid=grid,
        in_specs=[
            pl.BlockSpec((blk, D), lambda i: (i, 0)),
            pl.BlockSpec((blk, D), lambda i: (i, 0)),
            pl.BlockSpec((blk, 1), lambda i: (i, 0)),
            pl.BlockSpec((D, D), lambda i: (0, 0)),
            pl.BlockSpec((1, D), lambda i: (0, 0)),
        ],
        out_specs=pl.BlockSpec((blk, D), lambda i: (i, 0)),
    )(p0, p1, cnt_inv, wt, b)


def kernel(x, edge_index, edge_weight, W1, b1, W2, b2, W3, b3):
    row = edge_index[0]
    col = edge_index[1]
    zeros_deg = jnp.zeros((DEG_R, L), jnp.float32)
    deg_p = _sc_degree(col, zeros_deg)
    deg = (deg_p[0] + deg_p[1]).reshape(DEG_R * L)[:N]
    dis = jnp.where(deg > 0, jax.lax.rsqrt(jnp.maximum(deg, 1.0)), 0.0)
    cnt_inv = (1.0 / jnp.maximum(deg, 1.0))[:, None]
    w = _sc_edge_w(row, col, edge_weight, dis)

    zeros_nd = jnp.zeros((N, D), jnp.float32)
    h = x
    for (W, b, relu) in ((W1, b1, True), (W2, b2, True), (W3, b3, False)):
        parts = _sc_aggregate(h, row, col, w, zeros_nd)
        h = _tc_update(parts[0], parts[1], cnt_inv, W.T,
                       b.reshape(1, D), relu)
    return h


# speedup vs baseline: 3.2406x; 3.2406x over previous
import functools

import jax
import jax.numpy as jnp
from jax import lax
from jax.experimental import pallas as pl
from jax.experimental.pallas import tpu as pltpu, tpu_sc as plsc

NC = 2
NS = 16
L = 16
NW = NC * NS

N = 10000
E = 320000
D = 128

EPT = E // NW
G = EPT // L
N_PAD = 10240
NPT = N_PAD // NS
DEG_R = 640

_mesh = plsc.VectorSubcoreMesh(core_axis_name="c", subcore_axis_name="s")


@functools.partial(
    pl.kernel,
    out_type=jax.ShapeDtypeStruct((NW, N), jnp.float32),
    mesh=_mesh,
    compiler_params=pltpu.CompilerParams(needs_layout_passes=False),
    scratch_types=[
        pltpu.VMEM((EPT,), jnp.int32),
        pltpu.VMEM((N,), jnp.float32),
    ],
)
def _sc_degree3(col_hbm, zeros_hbm, out_hbm, col_v, deg_v):
    cid = lax.axis_index("c")
    sid = lax.axis_index("s")
    tid = sid * NC + cid
    pltpu.sync_copy(col_hbm.at[pl.ds(tid * EPT, EPT)], col_v)
    pltpu.sync_copy(zeros_hbm, deg_v)
    ones = jnp.ones((L,), jnp.float32)

    def body(g, carry):
        c16 = col_v[pl.ds(g * L, L)]
        plsc.addupdate_scatter(deg_v, [c16], ones)
        return carry

    lax.fori_loop(0, G, body, 0)
    pltpu.sync_copy(deg_v, out_hbm.at[tid])


@functools.partial(
    pl.kernel,
    out_type=jax.ShapeDtypeStruct((E,), jnp.float32),
    mesh=_mesh,
    compiler_params=pltpu.CompilerParams(needs_layout_passes=False),
    scratch_types=[
        pltpu.VMEM((N,), jnp.float32),
        pltpu.VMEM((EPT,), jnp.int32),
        pltpu.VMEM((EPT,), jnp.int32),
        pltpu.VMEM((EPT,), jnp.float32),
        pltpu.VMEM((EPT,), jnp.float32),
    ],
)
def _sc_edge_w(row_hbm, col_hbm, ew_hbm, dis_hbm, out_hbm,
               dis_v, row_v, col_v, ew_v, w_v):
    cid = lax.axis_index("c")
    sid = lax.axis_index("s")
    tid = sid * NC + cid
    pltpu.sync_copy(dis_hbm, dis_v)
    pltpu.sync_copy(row_hbm.at[pl.ds(tid * EPT, EPT)], row_v)
    pltpu.sync_copy(col_hbm.at[pl.ds(tid * EPT, EPT)], col_v)
    pltpu.sync_copy(ew_hbm.at[pl.ds(tid * EPT, EPT)], ew_v)

    def body(g, carry):
        sl = pl.ds(g * L, L)
        a = plsc.load_gather(dis_v, [row_v[sl]])
        b = plsc.load_gather(dis_v, [col_v[sl]])
        w_v[sl] = ew_v[sl] * (a * b)
        return carry

    lax.fori_loop(0, G, body, 0)
    pltpu.sync_copy(w_v, out_hbm.at[pl.ds(tid * EPT, EPT)])


FB = D // NW
CHUNK = 8000
NCHUNK = E // CHUNK
CG = CHUNK // L


@functools.partial(
    pl.kernel,
    out_type=jax.ShapeDtypeStruct((NW, N * FB), jnp.float32),
    mesh=_mesh,
    compiler_params=pltpu.CompilerParams(needs_layout_passes=False),
    scratch_types=[
        pltpu.VMEM((N * FB,), jnp.float32),
        pltpu.VMEM((N * FB,), jnp.float32),
        pltpu.VMEM((CHUNK,), jnp.int32),
        pltpu.VMEM((CHUNK,), jnp.int32),
        pltpu.VMEM((CHUNK,), jnp.float32),
    ],
)
def _sc_aggregate_cs(hcs_hbm, row_hbm, col_hbm, w_hbm, zeros_hbm, out_hbm,
                     h_v, acc_v, row_v, col_v, w_v):
    cid = lax.axis_index("c")
    sid = lax.axis_index("s")
    tid = sid * NC + cid
    pltpu.sync_copy(hcs_hbm.at[tid], h_v)
    pltpu.sync_copy(zeros_hbm, acc_v)

    fvecs = [jnp.full((L,), f, jnp.int32) for f in range(FB)]

    def chunk_body(c, carry):
        pltpu.sync_copy(row_hbm.at[pl.ds(c * CHUNK, CHUNK)], row_v)
        pltpu.sync_copy(col_hbm.at[pl.ds(c * CHUNK, CHUNK)], col_v)
        pltpu.sync_copy(w_hbm.at[pl.ds(c * CHUNK, CHUNK)], w_v)

        def body(g, carry2):
            sl = pl.ds(g * L, L)
            rb = row_v[sl] * FB
            cb = col_v[sl] * FB
            w16 = w_v[sl]
            for f in range(FB):
                vals = plsc.load_gather(h_v, [rb + fvecs[f]])
                plsc.addupdate_scatter(acc_v, [cb + fvecs[f]], vals * w16)
            return carry2

        lax.fori_loop(0, CG, body, 0)
        return carry

    lax.fori_loop(0, NCHUNK, chunk_body, 0)
    pltpu.sync_copy(acc_v, out_hbm.at[tid])


def _tc_update_body(p, cinv, w, b, out, *, relu):
    aggr = p[...] * cinv[...]
    y = jnp.dot(aggr, w[...], preferred_element_type=jnp.float32) + b[...]
    if relu:
        y = jnp.maximum(y, 0.0)
    out[...] = y


def _tc_update(p, cnt_inv, wt, b, relu):
    blk = 2000
    body = functools.partial(_tc_update_body, relu=relu)
    return pl.pallas_call(
        body,
        out_shape=jax.ShapeDtypeStruct((N, D), jnp.float32),
        grid=(N // blk,),
        in_specs=[
            pl.BlockSpec((blk, D), lambda i: (i, 0)),
            pl.BlockSpec((blk, 1), lambda i: (i, 0)),
            pl.BlockSpec((D, D), lambda i: (0, 0)),
            pl.BlockSpec((1, D), lambda i: (0, 0)),
        ],
        out_specs=pl.BlockSpec((blk, D), lambda i: (i, 0)),
    )(p, cnt_inv, wt, b)


def kernel(x, edge_index, edge_weight, W1, b1, W2, b2, W3, b3):
    row = edge_index[0]
    col = edge_index[1]
    zeros_n = jnp.zeros((N,), jnp.float32)
    deg = jnp.sum(_sc_degree3(col, zeros_n), axis=0)
    dis = jnp.where(deg > 0, 1.0 / jnp.sqrt(jnp.maximum(deg, 1.0)), 0.0)
    cnt_inv = (1.0 / jnp.maximum(deg, 1.0))[:, None]
    w = _sc_edge_w(row, col, edge_weight, dis)

    zeros_fb = jnp.zeros((N * FB,), jnp.float32)
    h = x
    for i, (W, b) in enumerate(((W1, b1), (W2, b2), (W3, b3))):
        last = i == 2
        hcs = h.reshape(N, NW, FB).transpose(1, 0, 2).reshape(NW, N * FB)
        out_cs = _sc_aggregate_cs(hcs, row, col, w, zeros_fb)
        summed = out_cs.reshape(NW, N, FB).transpose(1, 0, 2).reshape(N, D)
        h = _tc_update(summed, cnt_inv, W.T, b.reshape(1, D), relu=not last)
    return h


# speedup vs baseline: 3.6516x; 1.1268x over previous
import functools

import jax
import jax.numpy as jnp
from jax import lax
from jax.experimental import pallas as pl
from jax.experimental.pallas import tpu as pltpu, tpu_sc as plsc

NC = 2
NS = 16
L = 16
NW = NC * NS

N = 10000
E = 320000
D = 128

EPT = E // NW
G = EPT // L
N_PAD = 10240
NPT = N_PAD // NS
DEG_R = 640

_mesh = plsc.VectorSubcoreMesh(core_axis_name="c", subcore_axis_name="s")


@functools.partial(
    pl.kernel,
    out_type=jax.ShapeDtypeStruct((NW, N), jnp.float32),
    mesh=_mesh,
    compiler_params=pltpu.CompilerParams(needs_layout_passes=False),
    scratch_types=[
        pltpu.VMEM((EPT,), jnp.int32),
        pltpu.VMEM((N,), jnp.float32),
    ],
)
def _sc_degree3(col_hbm, zeros_hbm, out_hbm, col_v, deg_v):
    cid = lax.axis_index("c")
    sid = lax.axis_index("s")
    tid = sid * NC + cid
    pltpu.sync_copy(col_hbm.at[pl.ds(tid * EPT, EPT)], col_v)
    pltpu.sync_copy(zeros_hbm, deg_v)
    ones = jnp.ones((L,), jnp.float32)

    def body(g, carry):
        c16 = col_v[pl.ds(g * L, L)]
        plsc.addupdate_scatter(deg_v, [c16], ones)
        return carry

    lax.fori_loop(0, G, body, 0)
    pltpu.sync_copy(deg_v, out_hbm.at[tid])


@functools.partial(
    pl.kernel,
    out_type=jax.ShapeDtypeStruct((E,), jnp.float32),
    mesh=_mesh,
    compiler_params=pltpu.CompilerParams(needs_layout_passes=False),
    scratch_types=[
        pltpu.VMEM((N,), jnp.float32),
        pltpu.VMEM((EPT,), jnp.int32),
        pltpu.VMEM((EPT,), jnp.int32),
        pltpu.VMEM((EPT,), jnp.float32),
        pltpu.VMEM((EPT,), jnp.float32),
    ],
)
def _sc_edge_w(row_hbm, col_hbm, ew_hbm, dis_hbm, out_hbm,
               dis_v, row_v, col_v, ew_v, w_v):
    cid = lax.axis_index("c")
    sid = lax.axis_index("s")
    tid = sid * NC + cid
    pltpu.sync_copy(dis_hbm, dis_v)
    pltpu.sync_copy(row_hbm.at[pl.ds(tid * EPT, EPT)], row_v)
    pltpu.sync_copy(col_hbm.at[pl.ds(tid * EPT, EPT)], col_v)
    pltpu.sync_copy(ew_hbm.at[pl.ds(tid * EPT, EPT)], ew_v)

    def body(g, carry):
        sl = pl.ds(g * L, L)
        a = plsc.load_gather(dis_v, [row_v[sl]])
        b = plsc.load_gather(dis_v, [col_v[sl]])
        w_v[sl] = ew_v[sl] * (a * b)
        return carry

    lax.fori_loop(0, G, body, 0)
    pltpu.sync_copy(w_v, out_hbm.at[pl.ds(tid * EPT, EPT)])


FB = D // NW
CHUNK = 8000
NCHUNK = E // CHUNK
CG = CHUNK // L


@functools.partial(
    pl.kernel,
    out_type=jax.ShapeDtypeStruct((NW, N * FB), jnp.float32),
    mesh=_mesh,
    compiler_params=pltpu.CompilerParams(needs_layout_passes=False),
    scratch_types=[
        pltpu.VMEM((N * FB,), jnp.float32),
        pltpu.VMEM((N * FB,), jnp.float32),
        pltpu.VMEM((CHUNK,), jnp.int32),
        pltpu.VMEM((CHUNK,), jnp.int32),
        pltpu.VMEM((CHUNK,), jnp.float32),
        pltpu.VMEM((CHUNK,), jnp.int32),
        pltpu.VMEM((CHUNK,), jnp.int32),
        pltpu.VMEM((CHUNK,), jnp.float32),
        pltpu.SemaphoreType.DMA,
        pltpu.SemaphoreType.DMA,
    ],
)
def _sc_aggregate_cs(hcs_hbm, row_hbm, col_hbm, w_hbm, zeros_hbm, out_hbm,
                     h_v, acc_v, row_a, col_a, w_a, row_b, col_b, w_b,
                     sem_a, sem_b):
    cid = lax.axis_index("c")
    sid = lax.axis_index("s")
    tid = sid * NC + cid
    pltpu.sync_copy(hcs_hbm.at[tid], h_v)
    pltpu.sync_copy(zeros_hbm, acc_v)

    bufs = ((row_a, col_a, w_a, sem_a), (row_b, col_b, w_b, sem_b))
    fvecs = [jnp.full((L,), f, jnp.int32) for f in range(FB)]

    def issue(c, p):
        rv, cv, wv, sem = bufs[p]
        sl = pl.ds(c * CHUNK, CHUNK)
        pltpu.async_copy(row_hbm.at[sl], rv, sem)
        pltpu.async_copy(col_hbm.at[sl], cv, sem)
        pltpu.async_copy(w_hbm.at[sl], wv, sem)

    def drain(c, p):
        rv, cv, wv, sem = bufs[p]
        sl = pl.ds(c * CHUNK, CHUNK)
        pltpu.make_async_copy(row_hbm.at[sl], rv, sem).wait()
        pltpu.make_async_copy(col_hbm.at[sl], cv, sem).wait()
        pltpu.make_async_copy(w_hbm.at[sl], wv, sem).wait()

    def process(p):
        rv, cv, wv, _ = bufs[p]

        def body(g, carry2):
            sl = pl.ds(g * L, L)
            rb = rv[sl] * FB
            cb = cv[sl] * FB
            w16 = wv[sl]
            for f in range(FB):
                vals = plsc.load_gather(h_v, [rb + fvecs[f]])
                plsc.addupdate_scatter(acc_v, [cb + fvecs[f]], vals * w16)
            return carry2

        lax.fori_loop(0, CG, body, 0)

    issue(0, 0)

    def pair(i, carry):
        c0 = i * 2
        drain(c0, 0)

        @pl.when(c0 + 1 < NCHUNK)
        def _():
            issue(c0 + 1, 1)

        process(0)

        drain(c0 + 1, 1)

        @pl.when(c0 + 2 < NCHUNK)
        def _():
            issue(c0 + 2, 0)

        process(1)
        return carry

    lax.fori_loop(0, NCHUNK // 2, pair, 0)
    pltpu.sync_copy(acc_v, out_hbm.at[tid])


def _tc_update_body(p, cinv, w, b, out, *, relu):
    aggr = p[...] * cinv[...]
    y = jnp.dot(aggr, w[...], preferred_element_type=jnp.float32) + b[...]
    if relu:
        y = jnp.maximum(y, 0.0)
    out[...] = y


def _tc_update(p, cnt_inv, wt, b, relu):
    blk = 2000
    body = functools.partial(_tc_update_body, relu=relu)
    return pl.pallas_call(
        body,
        out_shape=jax.ShapeDtypeStruct((N, D), jnp.float32),
        grid=(N // blk,),
        in_specs=[
            pl.BlockSpec((blk, D), lambda i: (i, 0)),
            pl.BlockSpec((blk, 1), lambda i: (i, 0)),
            pl.BlockSpec((D, D), lambda i: (0, 0)),
            pl.BlockSpec((1, D), lambda i: (0, 0)),
        ],
        out_specs=pl.BlockSpec((blk, D), lambda i: (i, 0)),
    )(p, cnt_inv, wt, b)


def kernel(x, edge_index, edge_weight, W1, b1, W2, b2, W3, b3):
    row = edge_index[0]
    col = edge_index[1]
    zeros_n = jnp.zeros((N,), jnp.float32)
    deg = jnp.sum(_sc_degree3(col, zeros_n), axis=0)
    dis = jnp.where(deg > 0, 1.0 / jnp.sqrt(jnp.maximum(deg, 1.0)), 0.0)
    cnt_inv = (1.0 / jnp.maximum(deg, 1.0))[:, None]
    w = _sc_edge_w(row, col, edge_weight, dis)

    zeros_fb = jnp.zeros((N * FB,), jnp.float32)
    h = x
    for i, (W, b) in enumerate(((W1, b1), (W2, b2), (W3, b3))):
        last = i == 2
        hcs = h.reshape(N, NW, FB).transpose(1, 0, 2).reshape(NW, N * FB)
        out_cs = _sc_aggregate_cs(hcs, row, col, w, zeros_fb)
        summed = out_cs.reshape(NW, N, FB).transpose(1, 0, 2).reshape(N, D)
        h = _tc_update(summed, cnt_inv, W.T, b.reshape(1, D), relu=not last)
    return h
